# Initial kernel scaffold; baseline (speedup 1.0000x reference)
import jax
import jax.numpy as jnp
from jax.experimental import pallas as pl


def kernel(support_images, support_labels, query_images, W):
    raise NotImplementedError("write your pallas kernel here")


# speedup vs baseline: 10.1978x; 10.1978x over previous
import functools

import jax
import jax.numpy as jnp
from jax.experimental import pallas as pl
from jax.experimental.pallas import tpu as pltpu

K = 10
N_WAY = 5
NC = 8
D = 256
S = 4096
Q = 2048
QT = 256
POWER = 2
DELTA = 1.0


def _stage_a(support_ref, w_ref, labels_ref, zs_ref, onehot_t_ref, proto_ref, s2_ref):
    zs = jnp.dot(support_ref[...], w_ref[...], preferred_element_type=jnp.float32)
    zs_ref[...] = zs
    labels = labels_ref[...]
    classes = jax.lax.broadcasted_iota(jnp.int32, (NC, S), 0)
    onehot_t = (labels == classes).astype(jnp.float32)
    onehot_t_ref[...] = onehot_t
    counts = jnp.sum(onehot_t, axis=1, keepdims=True)
    proto_sums = jnp.dot(onehot_t, zs, preferred_element_type=jnp.float32)
    proto_ref[...] = proto_sums / jnp.maximum(counts, 1.0)
    s2_ref[...] = jnp.sum(zs * zs, axis=1, keepdims=True).reshape(1, S)


def _stage_b(q_ref, w_ref, zs_ref, onehot_t_ref, proto_ref, s2_ref, out_ref):
    zq = jnp.dot(q_ref[...], w_ref[...], preferred_element_type=jnp.float32)
    q2 = jnp.sum(zq * zq, axis=1, keepdims=True)
    cross = jax.lax.dot_general(zq, zs_ref[...], (((1,), (1,)), ((), ())),
                                preferred_element_type=jnp.float32)
    d2 = q2 + s2_ref[...] - 2.0 * cross

    col = jax.lax.broadcasted_iota(jnp.int32, (QT, S), 1)
    mode = jnp.zeros((QT, NC), dtype=jnp.float32)
    for _ in range(K):
        m = jnp.min(d2, axis=1, keepdims=True)
        eq = d2 == m
        idx = jnp.min(jnp.where(eq, col, S), axis=1, keepdims=True)
        sel = (col == idx).astype(jnp.float32)
        mode = mode + jax.lax.dot_general(sel, onehot_t_ref[...],
                                          (((1,), (1,)), ((), ())),
                                          preferred_element_type=jnp.float32)
        d2 = jnp.where(col == idx, jnp.inf, d2)

    proto = proto_ref[...]
    p2 = jnp.sum(proto * proto, axis=1, keepdims=True).reshape(1, NC)
    crossp = jax.lax.dot_general(zq, proto, (((1,), (1,)), ((), ())),
                                 preferred_element_type=jnp.float32)
    dp2 = q2 + p2 - 2.0 * crossp
    dists_p = jnp.sqrt(jnp.maximum(dp2, 1e-12))
    out_ref[...] = -(dists_p / (mode * mode + DELTA))


@jax.jit
def kernel(support_images, support_labels, query_images, W):
    labels2d = support_labels.reshape(1, S)

    zs, onehot_t, proto, s2 = pl.pallas_call(
        _stage_a,
        out_shape=[
            jax.ShapeDtypeStruct((S, D), jnp.float32),
            jax.ShapeDtypeStruct((NC, S), jnp.float32),
            jax.ShapeDtypeStruct((NC, D), jnp.float32),
            jax.ShapeDtypeStruct((1, S), jnp.float32),
        ],
    )(support_images, W, labels2d)

    grid = Q // QT
    scores = pl.pallas_call(
        _stage_b,
        grid=(grid,),
        in_specs=[
            pl.BlockSpec((QT, D), lambda i: (i, 0)),
            pl.BlockSpec((D, D), lambda i: (0, 0)),
            pl.BlockSpec((S, D), lambda i: (0, 0)),
            pl.BlockSpec((NC, S), lambda i: (0, 0)),
            pl.BlockSpec((NC, D), lambda i: (0, 0)),
            pl.BlockSpec((1, S), lambda i: (0, 0)),
        ],
        out_specs=pl.BlockSpec((QT, NC), lambda i: (i, 0)),
        out_shape=jax.ShapeDtypeStruct((Q, NC), jnp.float32),
    )(query_images, W, zs, onehot_t, proto, s2)

    return scores[:, :N_WAY]


# speedup vs baseline: 17.0775x; 1.6746x over previous
import functools

import jax
import jax.numpy as jnp
from jax.experimental import pallas as pl
from jax.experimental.pallas import tpu as pltpu

K = 10
N_WAY = 5
NC = 8
D = 256
S = 4096
Q = 2048
QT = 256
POWER = 2
DELTA = 1.0


def _stage_a(support_ref, w_ref, labels_ref, zs_ref, proto_ref, s2_ref):
    zs = jnp.dot(support_ref[...], w_ref[...], preferred_element_type=jnp.float32)
    zs_ref[...] = zs
    labels = labels_ref[...]
    classes = jax.lax.broadcasted_iota(jnp.int32, (NC, S), 0)
    onehot_t = (labels == classes).astype(jnp.float32)
    counts = jnp.sum(onehot_t, axis=1, keepdims=True)
    proto_sums = jnp.dot(onehot_t, zs, preferred_element_type=jnp.float32)
    proto_ref[...] = proto_sums / jnp.maximum(counts, 1.0)
    s2_ref[...] = jnp.sum(zs * zs, axis=1, keepdims=True).reshape(1, S)


def _stage_b(q_ref, w_ref, zs_ref, labels_ref, proto_ref, s2_ref, out_ref):
    zq = jnp.dot(q_ref[...], w_ref[...], preferred_element_type=jnp.float32)
    q2 = jnp.sum(zq * zq, axis=1, keepdims=True)
    cross = jax.lax.dot_general(zq, zs_ref[...], (((1,), (1,)), ((), ())),
                                preferred_element_type=jnp.float32)
    d2 = jnp.maximum(q2 + s2_ref[...] - 2.0 * cross, 0.0)

    bits = jax.lax.bitcast_convert_type(d2, jnp.int32)
    packed = (bits & ~jnp.int32(7)) | labels_ref[...]

    classes = jax.lax.broadcasted_iota(jnp.int32, (QT, NC), 1)
    mode = jnp.zeros((QT, NC), dtype=jnp.float32)
    for _ in range(K):
        m = jnp.min(packed, axis=1, keepdims=True)
        mode = mode + (classes == (m & 7)).astype(jnp.float32)
        packed = jnp.where(packed == m, jnp.int32(0x7FFFFFFF), packed)

    proto = proto_ref[...]
    p2 = jnp.sum(proto * proto, axis=1, keepdims=True).reshape(1, NC)
    crossp = jax.lax.dot_general(zq, proto, (((1,), (1,)), ((), ())),
                                 preferred_element_type=jnp.float32)
    dp2 = q2 + p2 - 2.0 * crossp
    dists_p = jnp.sqrt(jnp.maximum(dp2, 1e-12))
    out_ref[...] = -(dists_p / (mode * mode + DELTA))


@jax.jit
def kernel(support_images, support_labels, query_images, W):
    labels2d = support_labels.reshape(1, S)

    zs, proto, s2 = pl.pallas_call(
        _stage_a,
        out_shape=[
            jax.ShapeDtypeStruct((S, D), jnp.float32),
            jax.ShapeDtypeStruct((NC, D), jnp.float32),
            jax.ShapeDtypeStruct((1, S), jnp.float32),
        ],
    )(support_images, W, labels2d)

    grid = Q // QT
    scores = pl.pallas_call(
        _stage_b,
        grid=(grid,),
        in_specs=[
            pl.BlockSpec((QT, D), lambda i: (i, 0)),
            pl.BlockSpec((D, D), lambda i: (0, 0)),
            pl.BlockSpec((S, D), lambda i: (0, 0)),
            pl.BlockSpec((1, S), lambda i: (0, 0)),
            pl.BlockSpec((NC, D), lambda i: (0, 0)),
            pl.BlockSpec((1, S), lambda i: (0, 0)),
        ],
        out_specs=pl.BlockSpec((QT, NC), lambda i: (i, 0)),
        out_shape=jax.ShapeDtypeStruct((Q, NC), jnp.float32),
    )(query_images, W, zs, labels2d, proto, s2)

    return scores[:, :N_WAY]
